# baseline (speedup 1.0000x reference)
import functools

import jax
import jax.numpy as jnp
from jax import lax
from jax.experimental import pallas as pl
from jax.experimental.pallas import tpu as pltpu
from jax.experimental.pallas import tpu_sc as plsc

_THRESHOLD = 0.8
_S = 4096
_C = 2048
_L = 16
_NC = 2
_NS = 16
_NW = _NC * _NS

_R_SC = 1024
_SC_BASE = _S - _R_SC
_R_TC = _SC_BASE
_ROWS_PER_W = _R_SC // _NW
_CH = 16
_NCHUNK = _ROWS_PER_W // _CH
_NBUF = 2
_NPOLY = 12

_TC_BLOCK = 128


def _sc_body(x_hbm, out0_hbm, out1_hbm, buf, part, sem0, sem1):
    cid = lax.axis_index("c")
    sid = lax.axis_index("s")
    wid = sid * _NC + cid
    base = _SC_BASE + wid * _ROWS_PER_W

    sems = (sem0, sem1)

    def start(k):
        return pltpu.async_copy(
            x_hbm.at[0, pl.ds(base + k * _CH, _CH), :],
            buf.at[k % _NBUF],
            sems[k % _NBUF],
        )

    row_iota = lax.iota(jnp.int32, _L)
    loss_acc = jnp.zeros((_L,), jnp.float32)
    count_acc = jnp.zeros((_L,), jnp.float32)

    pending = [start(k) for k in range(min(_NBUF, _NCHUNK))]
    for k in range(_NCHUNK):
        b = k % _NBUF
        if k + _NBUF < _NCHUNK:
            pending.append(start(k + _NBUF))
        pending.pop(0).wait()

        accs = tuple(
            jnp.maximum(buf[b, r, pl.ds(0, _L)], buf[b, r, pl.ds(_L, _L)])
            for r in range(_CH)
        )

        def col_body(i, a):
            base_c = i * (2 * _L)
            a = tuple(
                jnp.maximum(a[r], buf[b, r, pl.ds(base_c, _L)])
                for r in range(_CH)
            )
            return tuple(
                jnp.maximum(a[r], buf[b, r, pl.ds(base_c + _L, _L)])
                for r in range(_CH)
            )

        accs = lax.fori_loop(1, _C // (2 * _L), col_body, accs)

        p_vec = jnp.zeros((_L,), jnp.float32)
        for r in range(_CH):
            m_r = jnp.max(accs[r])
            p_vec = jnp.where(row_iota == jnp.int32(r), m_r, p_vec)

        mask = p_vec > jnp.float32(_THRESHOLD)
        q = jnp.where(mask, jnp.float32(1.0) - p_vec, jnp.float32(0.0))
        poly = jnp.full((_L,), jnp.float32(1.0 / _NPOLY))
        for k_ in range(_NPOLY - 1, 0, -1):
            poly = poly * q + jnp.float32(1.0 / k_)
        neg_log_p = q * poly
        loss_acc = loss_acc + q * q * neg_log_p
        count_acc = count_acc + jnp.where(
            mask, jnp.float32(1.0), jnp.float32(0.0)
        )

    part[0, :] = loss_acc
    part[1, :] = count_acc

    @pl.when(cid == 0)
    def _():
        pltpu.sync_copy(part.at[0], out0_hbm.at[0, pl.ds(sid * _L, _L)])
        pltpu.sync_copy(part.at[1], out0_hbm.at[1, pl.ds(sid * _L, _L)])

    @pl.when(cid == 1)
    def _():
        pltpu.sync_copy(part.at[0], out1_hbm.at[0, pl.ds(sid * _L, _L)])
        pltpu.sync_copy(part.at[1], out1_hbm.at[1, pl.ds(sid * _L, _L)])


@functools.cache
def _make_sc_call():
    return pl.kernel(
        _sc_body,
        out_type=(
            jax.ShapeDtypeStruct((2, _NS * _L), jnp.float32),
            jax.ShapeDtypeStruct((2, _NS * _L), jnp.float32),
        ),
        mesh=plsc.VectorSubcoreMesh(
            core_axis_name="c", subcore_axis_name="s",
            num_cores=_NC, num_subcores=_NS,
        ),
        compiler_params=pltpu.CompilerParams(needs_layout_passes=False),
        scratch_types=[
            pltpu.VMEM((_NBUF, _CH, _C), jnp.float32),
            pltpu.VMEM((2, _L), jnp.float32),
            pltpu.SemaphoreType.DMA,
            pltpu.SemaphoreType.DMA,
        ],
    )


def _tc_part_body(x_ref, lo_ref, co_ref):
    x = x_ref[0].reshape(_TC_BLOCK // 128, 128, _C)
    p = jnp.max(x, axis=2)
    mask = p > jnp.float32(_THRESHOLD)
    safe_p = jnp.where(mask, p, jnp.float32(1.0))
    q = jnp.float32(1.0) - safe_p
    loss = q * q * (-jnp.log(safe_p))

    @pl.when(pl.program_id(0) == 0)
    def _():
        lo_ref[...] = jnp.zeros_like(lo_ref)
        co_ref[...] = jnp.zeros_like(co_ref)

    lo_ref[...] += loss
    co_ref[...] += mask.astype(jnp.float32)


def _tc_part(inputs, row_start, nrows):
    grid = nrows // _TC_BLOCK
    nsub = _TC_BLOCK // 128
    shape = jax.ShapeDtypeStruct((nsub, 128), jnp.float32)
    return pl.pallas_call(
        _tc_part_body,
        grid=(grid,),
        in_specs=[
            pl.BlockSpec(
                (1, _TC_BLOCK, _C),
                lambda i: (0, i + row_start // _TC_BLOCK, 0),
            )
        ],
        out_specs=(
            pl.BlockSpec((nsub, 128), lambda i: (0, 0)),
            pl.BlockSpec((nsub, 128), lambda i: (0, 0)),
        ),
        out_shape=(shape, shape),
        compiler_params=pltpu.CompilerParams(
            dimension_semantics=("arbitrary",)
        ),
    )(inputs)


def _finish_body(sc0_ref, sc1_ref, tl_ref, tc_ref, o_ref):
    loss_sum = (
        jnp.sum(sc0_ref[0, :])
        + jnp.sum(sc1_ref[0, :])
        + jnp.sum(tl_ref[...])
    )
    count = (
        jnp.sum(sc0_ref[1, :])
        + jnp.sum(sc1_ref[1, :])
        + jnp.sum(tc_ref[...])
    )
    val = loss_sum / jnp.maximum(count, jnp.float32(1.0))
    o_ref[...] = jnp.reshape(val, (1, 1))


def kernel(inputs):
    sc0, sc1 = _make_sc_call()(inputs)
    t_loss, t_cnt = _tc_part(inputs, 0, _R_TC)
    out = pl.pallas_call(
        _finish_body,
        out_shape=jax.ShapeDtypeStruct((1, 1), jnp.float32),
    )(sc0, sc1, t_loss, t_cnt)
    return out[0, 0]


# speedup vs baseline: 1.2484x; 1.2484x over previous
import functools

import jax
import jax.numpy as jnp
from jax import lax
from jax.experimental import pallas as pl
from jax.experimental.pallas import tpu as pltpu
from jax.experimental.pallas import tpu_sc as plsc

_THRESHOLD = 0.8
_S = 4096
_C = 2048
_L = 16
_NC = 2
_NS = 16
_NW = _NC * _NS

_R_SC = 1024
_SC_BASE = _S - _R_SC
_R_TC = _SC_BASE
_ROWS_PER_W = _R_SC // _NW
_CH = 16
_NCHUNK = _ROWS_PER_W // _CH
_NBUF = 2
_NPOLY = 12

_TC_BLOCK = 512


def _sc_body(x_hbm, out0_hbm, out1_hbm, buf, part, sem0, sem1):
    cid = lax.axis_index("c")
    sid = lax.axis_index("s")
    wid = sid * _NC + cid
    base = _SC_BASE + wid * _ROWS_PER_W

    sems = (sem0, sem1)

    def start(k):
        return pltpu.async_copy(
            x_hbm.at[0, pl.ds(base + k * _CH, _CH), :],
            buf.at[k % _NBUF],
            sems[k % _NBUF],
        )

    row_iota = lax.iota(jnp.int32, _L)
    loss_acc = jnp.zeros((_L,), jnp.float32)
    count_acc = jnp.zeros((_L,), jnp.float32)

    pending = [start(k) for k in range(min(_NBUF, _NCHUNK))]
    for k in range(_NCHUNK):
        b = k % _NBUF
        if k + _NBUF < _NCHUNK:
            pending.append(start(k + _NBUF))
        pending.pop(0).wait()

        accs = tuple(
            jnp.maximum(buf[b, r, pl.ds(0, _L)], buf[b, r, pl.ds(_L, _L)])
            for r in range(_CH)
        )

        def col_body(i, a):
            base_c = i * (2 * _L)
            a = tuple(
                jnp.maximum(a[r], buf[b, r, pl.ds(base_c, _L)])
                for r in range(_CH)
            )
            return tuple(
                jnp.maximum(a[r], buf[b, r, pl.ds(base_c + _L, _L)])
                for r in range(_CH)
            )

        accs = lax.fori_loop(1, _C // (2 * _L), col_body, accs)

        p_vec = jnp.zeros((_L,), jnp.float32)
        for r in range(_CH):
            m_r = jnp.max(accs[r])
            p_vec = jnp.where(row_iota == jnp.int32(r), m_r, p_vec)

        mask = p_vec > jnp.float32(_THRESHOLD)
        q = jnp.where(mask, jnp.float32(1.0) - p_vec, jnp.float32(0.0))
        poly = jnp.full((_L,), jnp.float32(1.0 / _NPOLY))
        for k_ in range(_NPOLY - 1, 0, -1):
            poly = poly * q + jnp.float32(1.0 / k_)
        neg_log_p = q * poly
        loss_acc = loss_acc + q * q * neg_log_p
        count_acc = count_acc + jnp.where(
            mask, jnp.float32(1.0), jnp.float32(0.0)
        )

    part[0, :] = loss_acc
    part[1, :] = count_acc

    @pl.when(cid == 0)
    def _():
        pltpu.sync_copy(part.at[0], out0_hbm.at[0, pl.ds(sid * _L, _L)])
        pltpu.sync_copy(part.at[1], out0_hbm.at[1, pl.ds(sid * _L, _L)])

    @pl.when(cid == 1)
    def _():
        pltpu.sync_copy(part.at[0], out1_hbm.at[0, pl.ds(sid * _L, _L)])
        pltpu.sync_copy(part.at[1], out1_hbm.at[1, pl.ds(sid * _L, _L)])


@functools.cache
def _make_sc_call():
    return pl.kernel(
        _sc_body,
        out_type=(
            jax.ShapeDtypeStruct((2, _NS * _L), jnp.float32),
            jax.ShapeDtypeStruct((2, _NS * _L), jnp.float32),
        ),
        mesh=plsc.VectorSubcoreMesh(
            core_axis_name="c", subcore_axis_name="s",
            num_cores=_NC, num_subcores=_NS,
        ),
        compiler_params=pltpu.CompilerParams(needs_layout_passes=False),
        scratch_types=[
            pltpu.VMEM((_NBUF, _CH, _C), jnp.float32),
            pltpu.VMEM((2, _L), jnp.float32),
            pltpu.SemaphoreType.DMA,
            pltpu.SemaphoreType.DMA,
        ],
    )


def _tc_part_body(x_ref, lo_ref, co_ref):
    x = x_ref[0].reshape(_TC_BLOCK // 128, 128, _C)
    p = jnp.max(x, axis=2)
    mask = p > jnp.float32(_THRESHOLD)
    safe_p = jnp.where(mask, p, jnp.float32(1.0))
    q = jnp.float32(1.0) - safe_p
    loss = q * q * (-jnp.log(safe_p))

    @pl.when(pl.program_id(0) == 0)
    def _():
        lo_ref[...] = jnp.zeros_like(lo_ref)
        co_ref[...] = jnp.zeros_like(co_ref)

    lo_ref[...] += loss
    co_ref[...] += mask.astype(jnp.float32)


def _tc_part(inputs, row_start, nrows):
    grid = nrows // _TC_BLOCK
    nsub = _TC_BLOCK // 128
    shape = jax.ShapeDtypeStruct((nsub, 128), jnp.float32)
    return pl.pallas_call(
        _tc_part_body,
        grid=(grid,),
        in_specs=[
            pl.BlockSpec(
                (1, _TC_BLOCK, _C),
                lambda i: (0, i + row_start // _TC_BLOCK, 0),
            )
        ],
        out_specs=(
            pl.BlockSpec((nsub, 128), lambda i: (0, 0)),
            pl.BlockSpec((nsub, 128), lambda i: (0, 0)),
        ),
        out_shape=(shape, shape),
        compiler_params=pltpu.CompilerParams(
            dimension_semantics=("arbitrary",)
        ),
    )(inputs)


def _finish_body(sc0_ref, sc1_ref, tl_ref, tc_ref, o_ref):
    loss_sum = (
        jnp.sum(sc0_ref[0, :])
        + jnp.sum(sc1_ref[0, :])
        + jnp.sum(tl_ref[...])
    )
    count = (
        jnp.sum(sc0_ref[1, :])
        + jnp.sum(sc1_ref[1, :])
        + jnp.sum(tc_ref[...])
    )
    val = loss_sum / jnp.maximum(count, jnp.float32(1.0))
    o_ref[...] = jnp.reshape(val, (1, 1))


def kernel(inputs):
    sc0, sc1 = _make_sc_call()(inputs)
    t_loss, t_cnt = _tc_part(inputs, 0, _R_TC)
    out = pl.pallas_call(
        _finish_body,
        out_shape=jax.ShapeDtypeStruct((1, 1), jnp.float32),
    )(sc0, sc1, t_loss, t_cnt)
    return out[0, 0]
